# Initial kernel scaffold; baseline (speedup 1.0000x reference)
import jax
import jax.numpy as jnp
from jax.experimental import pallas as pl


def kernel(x, edge_index, W1l, b1, W1r, W2l, b2, W2r):
    raise NotImplementedError("write your pallas kernel here")


# speedup vs baseline: 6.2325x; 6.2325x over previous
import functools

import jax
import jax.numpy as jnp
from jax import lax
from jax.experimental import pallas as pl
from jax.experimental.pallas import tpu as pltpu
from jax.experimental.pallas import tpu_sc as plsc

N_NODES = 10000
N_EDGES = 320000
IN_FEATS = 128
HIDDEN = 64
OUT_FEATS = 32

NP = 10240
NW = 32
ROWS_PER_W = 80
EP = NW * ROWS_PER_W * 128
ROWS_PER_TILE = NP // 16
BLK = 1024
GRID = NP // BLK


def _make_sc_agg(feat, with_count):
    out_type = [jax.ShapeDtypeStruct((2 * NP, feat), jnp.float32)]
    if with_count:
        out_type.append(jax.ShapeDtypeStruct((2 * NP,), jnp.float32))

    scratch = dict(
        src_v=pltpu.VMEM((ROWS_PER_W, 128), jnp.int32),
        dst_v=pltpu.VMEM((ROWS_PER_W, 128), jnp.int32),
        rows_v=pltpu.VMEM((512, feat), jnp.float32),
        zbuf=pltpu.VMEM((64, feat), jnp.float32),
        acc_sh=pltpu.VMEM_SHARED((NP, feat), jnp.float32),
        sem=pltpu.SemaphoreType.DMA,
    )
    if with_count:
        scratch.update(
            zbuf1=pltpu.VMEM((ROWS_PER_TILE,), jnp.float32),
            ones_v=pltpu.VMEM((128,), jnp.float32),
            cnt_sh=pltpu.VMEM_SHARED((NP,), jnp.float32),
        )

    mesh = plsc.VectorSubcoreMesh(
        core_axis_name="c", subcore_axis_name="s", num_cores=2, num_subcores=16
    )

    def body(rows_hbm, src_hbm, dst_hbm, *refs):
        if with_count:
            agg_out, cnt_out, src_v, dst_v, rows_v, zbuf, acc_sh, sem, \
                zbuf1, ones_v, cnt_sh = refs
        else:
            agg_out, src_v, dst_v, rows_v, zbuf, acc_sh, sem = refs
        c = lax.axis_index("c")
        s = lax.axis_index("s")

        z16 = jnp.zeros((16,), jnp.float32)

        def zrow(i, carry):
            for g in range(feat // 16):
                zbuf[i, pl.ds(g * 16, 16)] = z16
            return carry
        lax.fori_loop(0, 64, zrow, 0)

        def zslice(k, carry):
            pltpu.sync_copy(zbuf, acc_sh.at[pl.ds(s * ROWS_PER_TILE + k * 64, 64)])
            return carry
        lax.fori_loop(0, ROWS_PER_TILE // 64, zslice, 0)

        if with_count:
            def zc(i, carry):
                zbuf1[pl.ds(i * 16, 16)] = z16
                return carry
            lax.fori_loop(0, ROWS_PER_TILE // 16, zc, 0)
            pltpu.sync_copy(zbuf1, cnt_sh.at[pl.ds(s * ROWS_PER_TILE, ROWS_PER_TILE)])
            o16 = jnp.full((16,), 1.0, jnp.float32)
            for g in range(8):
                ones_v[pl.ds(g * 16, 16)] = o16

        plsc.subcore_barrier()

        wrow = (c * 16 + s) * ROWS_PER_W
        pltpu.sync_copy(src_hbm.at[pl.ds(wrow, ROWS_PER_W)], src_v)
        pltpu.sync_copy(dst_hbm.at[pl.ds(wrow, ROWS_PER_W)], dst_v)

        def blk(j, carry):
            for u in range(4):
                r = j * 4 + u
                g = pltpu.async_copy(
                    rows_hbm.at[src_v.at[r]], rows_v.at[pl.ds(u * 128, 128)], sem
                )
                g.wait()
                pltpu.sync_copy(
                    rows_v.at[pl.ds(u * 128, 128)], acc_sh.at[dst_v.at[r]], add=True
                )
                if with_count:
                    pltpu.sync_copy(ones_v, cnt_sh.at[dst_v.at[r]], add=True)
            return carry
        lax.fori_loop(0, ROWS_PER_W // 4, blk, 0)

        plsc.subcore_barrier()

        off = c * NP + s * ROWS_PER_TILE
        pltpu.sync_copy(
            acc_sh.at[pl.ds(s * ROWS_PER_TILE, ROWS_PER_TILE)],
            agg_out.at[pl.ds(off, ROWS_PER_TILE)],
        )
        if with_count:
            pltpu.sync_copy(
                cnt_sh.at[pl.ds(s * ROWS_PER_TILE, ROWS_PER_TILE)],
                cnt_out.at[pl.ds(off, ROWS_PER_TILE)],
            )

    return pl.kernel(
        body, out_type=out_type, mesh=mesh,
        scratch_types=list(scratch.values()),
        compiler_params=pltpu.CompilerParams(use_tc_tiling_on_sc=False),
    )


_sc_agg1 = _make_sc_agg(HIDDEN, True)
_sc_agg2 = _make_sc_agg(OUT_FEATS, False)


def _mm_kernel(x_ref, w_ref, o_ref):
    o_ref[...] = lax.dot_general(
        x_ref[...], w_ref[...], (((1,), (1,)), ((), ())),
        preferred_element_type=jnp.float32,
    )


def _project(x, w):
    k = x.shape[1]
    f = w.shape[0]
    return pl.pallas_call(
        _mm_kernel,
        grid=(GRID,),
        in_specs=[
            pl.BlockSpec((BLK, k), lambda i: (i, 0)),
            pl.BlockSpec((f, k), lambda i: (0, 0)),
        ],
        out_specs=pl.BlockSpec((BLK, f), lambda i: (i, 0)),
        out_shape=jax.ShapeDtypeStruct((NP, f), jnp.float32),
    )(x, w)


def _epilogue_kernel(relu, a0_ref, a1_ref, c0_ref, c1_ref, x_ref, wr_ref,
                     b_ref, h_ref):
    cnt = jnp.maximum(c0_ref[...] + c1_ref[...], 1.0)
    mean = (a0_ref[...] + a1_ref[...]) / cnt
    h = mean + b_ref[...] + lax.dot_general(
        x_ref[...], wr_ref[...], (((1,), (1,)), ((), ())),
        preferred_element_type=jnp.float32,
    )
    if relu:
        h = jnp.maximum(h, 0.0)
    h_ref[...] = h


def _epilogue(agg, cnt, x, wr, b, relu):
    f = wr.shape[0]
    k = x.shape[1]
    a0, a1 = agg[:NP], agg[NP:]
    c0 = cnt[:NP].reshape(NP, 1)
    c1 = cnt[NP:].reshape(NP, 1)
    b2 = b.reshape(1, f)
    return pl.pallas_call(
        functools.partial(_epilogue_kernel, relu),
        grid=(GRID,),
        in_specs=[
            pl.BlockSpec((BLK, f), lambda i: (i, 0)),
            pl.BlockSpec((BLK, f), lambda i: (i, 0)),
            pl.BlockSpec((BLK, 1), lambda i: (i, 0)),
            pl.BlockSpec((BLK, 1), lambda i: (i, 0)),
            pl.BlockSpec((BLK, k), lambda i: (i, 0)),
            pl.BlockSpec((f, k), lambda i: (0, 0)),
            pl.BlockSpec((1, f), lambda i: (0, 0)),
        ],
        out_specs=pl.BlockSpec((BLK, f), lambda i: (i, 0)),
        out_shape=jax.ShapeDtypeStruct((NP, f), jnp.float32),
    )(a0, a1, c0, c1, x, wr, b2)


def kernel(x, edge_index, W1l, b1, W1r, W2l, b2, W2r):
    src = edge_index[0].astype(jnp.int32)
    dst = edge_index[1].astype(jnp.int32)
    pad = EP - N_EDGES
    src_p = jnp.concatenate([src, jnp.zeros((pad,), jnp.int32)]).reshape(-1, 128)
    dst_p = jnp.concatenate([dst, jnp.full((pad,), NP - 1, jnp.int32)]).reshape(-1, 128)
    x_p = jnp.pad(x, ((0, NP - N_NODES), (0, 0)))

    p1 = _project(x_p, W1l)
    agg1, cnt = _sc_agg1(p1, src_p, dst_p)
    h = _epilogue(agg1, cnt, x_p, W1r, b1, True)
    p2 = _project(h, W2l)
    agg2 = _sc_agg2(p2, src_p, dst_p)[0]
    out = _epilogue(agg2, cnt, h, W2r, b2, False)
    return out[:N_NODES]
